# speedup vs baseline: 1.9452x; 1.9452x over previous
import jax
import jax.numpy as jnp
from jax import lax
from jax.experimental import pallas as pl
from jax.experimental.pallas import tpu as pltpu
from jax.experimental.pallas import tpu_sc as plsc

NUM_CORES = 2
NUM_SUBCORES = 16
LANES = 16
TILES = NUM_CORES * NUM_SUBCORES

_MESH = dict(core_axis_name="c", subcore_axis_name="s")
_CP = pltpu.CompilerParams(needs_layout_passes=False, use_tc_tiling_on_sc=False)

N_DUMMY = 128


def _wid():
    return lax.axis_index("s") * NUM_CORES + lax.axis_index("c")


def _iota16():
    return lax.iota(jnp.int32, LANES)


def _copy_body(in_ref, out_ref):
    out_ref[...] = in_ref[...]


def _k1_copy(emb):
    rows = emb.shape[0]
    blk = 10000
    assert rows % blk == 0
    return pl.pallas_call(
        _copy_body,
        out_shape=jax.ShapeDtypeStruct(emb.shape, emb.dtype),
        in_specs=[pl.BlockSpec((blk, emb.shape[1]), lambda i: (i, 0))],
        out_specs=pl.BlockSpec((blk, emb.shape[1]), lambda i: (i, 0)),
        grid=(rows // blk,),
    )(emb)


def _make_k2(B, D, bpt, nchunk):
    nb16 = bpt // LANES

    @pl.kernel(
        out_type=(
            jax.ShapeDtypeStruct((B, D), jnp.float32),
            jax.ShapeDtypeStruct((B,), jnp.int32),
        ),
        mesh=plsc.VectorSubcoreMesh(**_MESH),
        scratch_types=[
            pltpu.VMEM((bpt,), jnp.int32),
            pltpu.VMEM((bpt,), jnp.int32),
            pltpu.VMEM((nchunk, 128), jnp.int32),
            pltpu.VMEM((bpt,), jnp.int32),
            pltpu.VMEM((bpt,), jnp.int32),
            pltpu.VMEM((nchunk, 128), jnp.int32),
            pltpu.VMEM((bpt, D), jnp.float32),
            pltpu.VMEM((bpt, D), jnp.float32),
            pltpu.SemaphoreType.DMA,
        ],
        compiler_params=_CP,
    )
    def k2(t_hbm, eidx_hbm, cnw_hbm, emb_hbm, x_hbm, out_hbm, eout_hbm,
           t_v, e_v, widx_v, cw_v, ic_v, safe_v, rows_v, x_v, sem):
        w = _wid()
        base = w * bpt
        pltpu.sync_copy(t_hbm.at[pl.ds(base, bpt)], t_v)
        pltpu.async_copy(x_hbm.at[pl.ds(base, bpt)], x_v, sem).wait()

        @pl.loop(0, nb16)
        def _(c):
            t16 = t_v[pl.ds(c * LANES, LANES)]
            ci = c // 8
            cl = (c % 8) * LANES
            widx_v.at[ci, pl.ds(cl, LANES)][...] = t16 >> 2

        for c in range(nchunk):
            pltpu.async_copy(eidx_hbm.at[t_v.at[pl.ds(c * 128, 128)]],
                             e_v.at[pl.ds(c * 128, 128)], sem).wait()
            pltpu.async_copy(cnw_hbm.at[widx_v.at[c]],
                             cw_v.at[pl.ds(c * 128, 128)], sem).wait()

        @pl.loop(0, nb16)
        def _(c):
            off = c * LANES
            t16 = t_v[pl.ds(off, LANES)]
            e16 = e_v[pl.ds(off, LANES)]
            cw16 = cw_v[pl.ds(off, LANES)]
            ic = (cw16 >> ((t16 & 3) * 8)) & 0xFF
            bglob = base + off + _iota16()
            safe = jnp.where(ic != 0, jnp.maximum(e16, 0), bglob & 0x3FFF)
            ic_v.at[pl.ds(off, LANES)][...] = ic
            ci = c // 8
            cl = (c % 8) * LANES
            safe_v.at[ci, pl.ds(cl, LANES)][...] = safe

        for c in range(nchunk):
            pltpu.async_copy(emb_hbm.at[safe_v.at[c]],
                             rows_v.at[pl.ds(c * 128, 128)], sem).wait()

        @pl.loop(0, bpt)
        def _(r):
            m = plsc.load_gather(ic_v, [jnp.full((LANES,), r, jnp.int32)]) != 0
            for l in range(D // LANES):
                sl = pl.ds(l * LANES, LANES)
                rows_v.at[r, sl][...] = jnp.where(m, rows_v.at[r, sl][...],
                                                  x_v.at[r, sl][...])

        pltpu.sync_copy(rows_v, out_hbm.at[pl.ds(base, bpt)])
        pltpu.sync_copy(e_v, eout_hbm.at[pl.ds(base, bpt)])

    return k2


def _make_kw(B, own, tag_len):
    nchunks = B // LANES

    @pl.kernel(
        out_type=jax.ShapeDtypeStruct((tag_len,), jnp.int32),
        mesh=plsc.VectorSubcoreMesh(**_MESH),
        scratch_types=[
            pltpu.VMEM((B,), jnp.int32),
            pltpu.VMEM((own,), jnp.int32),
        ],
        compiler_params=_CP,
    )
    def kw(eall_hbm, tag_hbm, e_v, tag_v):
        w = _wid()
        base = w * own
        pltpu.sync_copy(eall_hbm, e_v)

        @pl.loop(0, own // LANES)
        def _(c):
            tag_v.at[pl.ds(c * LANES, LANES)][...] = jnp.full(
                (LANES,), -1, jnp.int32)

        @pl.loop(0, nchunks)
        def _(c):
            e16 = e_v[pl.ds(c * LANES, LANES)]
            b16 = c * LANES + _iota16()
            mask = (e16 >= base) & (e16 < base + own)
            off = jnp.where(mask, e16 - base, 0)
            plsc.store_scatter(tag_v, [off], b16, mask=mask)

        pltpu.sync_copy(tag_v, tag_hbm.at[pl.ds(base, own)])

    return kw


def _make_kf(B, nwords_p, wpt):
    nchunks = B // LANES

    @pl.kernel(
        out_type=jax.ShapeDtypeStruct((nwords_p,), jnp.int32),
        mesh=plsc.VectorSubcoreMesh(**_MESH),
        scratch_types=[
            pltpu.VMEM((wpt,), jnp.int32),
            pltpu.VMEM((B,), jnp.int32),
            pltpu.VMEM((B,), jnp.int32),
        ],
        compiler_params=_CP,
    )
    def kf(cnw_hbm, t_hbm, eall_hbm, ncw_hbm, w_v, t_v, e_v):
        w = _wid()
        base = w * wpt
        pltpu.sync_copy(cnw_hbm.at[pl.ds(base, wpt)], w_v)
        pltpu.sync_copy(t_hbm, t_v)
        pltpu.sync_copy(eall_hbm, e_v)

        @pl.loop(0, nchunks)
        def _(c):
            t16 = t_v[pl.ds(c * LANES, LANES)]
            e16 = e_v[pl.ds(c * LANES, LANES)]
            wi = t16 >> 2
            mask = (e16 != -1) & (wi >= base) & (wi < base + wpt)
            off = jnp.where(mask, wi - base, 0)
            shift = (t16 & 3) * 8
            bit = jnp.where(mask, 1 << shift, 0)

            def cond(carry):
                return jnp.any(carry)

            def body(carry):
                g = plsc.load_gather(w_v, [off])
                plsc.store_scatter(w_v, [off], g | bit, mask=carry)
                g2 = plsc.load_gather(w_v, [off])
                return carry & ((g2 & bit) != bit)

            lax.while_loop(cond, body, mask)

        pltpu.sync_copy(w_v, ncw_hbm.at[pl.ds(base, wpt)])

    return kf


def _make_k3(B, D, bpt, nchunk):
    nb16 = bpt // LANES

    @pl.kernel(
        mesh=plsc.VectorSubcoreMesh(**_MESH),
        scratch_types=[
            pltpu.VMEM((bpt,), jnp.int32),
            pltpu.VMEM((nchunk, 128), jnp.int32),
            pltpu.VMEM((bpt,), jnp.int32),
            pltpu.VMEM((nchunk, 128), jnp.int32),
            pltpu.VMEM((bpt, D), jnp.float32),
            pltpu.SemaphoreType.DMA,
        ],
        compiler_params=_CP,
    )
    def k3(eall_hbm, tag_hbm, x_hbm, emb_ref,
           e_v, gidx_v, tg_v, sidx_v, x_v, sem):
        w = _wid()
        base = w * bpt
        pltpu.sync_copy(eall_hbm.at[pl.ds(base, bpt)], e_v)
        pltpu.async_copy(x_hbm.at[pl.ds(base, bpt)], x_v, sem).wait()

        @pl.loop(0, nb16)
        def _(c):
            e16 = e_v[pl.ds(c * LANES, LANES)]
            bglob = base + c * LANES + _iota16()
            gidx = jnp.where(e16 != -1, e16, bglob)
            ci = c // 8
            cl = (c % 8) * LANES
            gidx_v.at[ci, pl.ds(cl, LANES)][...] = gidx

        for c in range(nchunk):
            pltpu.async_copy(tag_hbm.at[gidx_v.at[c]],
                             tg_v.at[pl.ds(c * 128, 128)], sem).wait()

        @pl.loop(0, nb16)
        def _(c):
            off = c * LANES
            e16 = e_v[pl.ds(off, LANES)]
            tg16 = tg_v[pl.ds(off, LANES)]
            bglob = base + off + _iota16()
            winner = (e16 != -1) & (tg16 == bglob)
            sidx = jnp.where(winner, e16, bglob & (N_DUMMY - 1))
            ci = c // 8
            cl = (c % 8) * LANES
            sidx_v.at[ci, pl.ds(cl, LANES)][...] = sidx

        for c in range(nchunk):
            pltpu.async_copy(x_v.at[pl.ds(c * 128, 128)],
                             emb_ref.at[sidx_v.at[c]], sem).wait()

    return k3


def _make_k4(D):
    spt = N_DUMMY // 8

    @pl.kernel(
        mesh=plsc.VectorSubcoreMesh(**_MESH),
        scratch_types=[
            pltpu.VMEM((LANES,), jnp.int32),
            pltpu.VMEM((LANES,), jnp.int32),
            pltpu.VMEM((LANES, D), jnp.float32),
            pltpu.VMEM((LANES, D), jnp.float32),
            pltpu.SemaphoreType.DMA,
        ],
        compiler_params=_CP,
    )
    def k4(tag_hbm, x_hbm, emb_in_hbm, emb_ref, tg_v, widx_v, xg_v, eg_v, sem):
        w = _wid()

        @pl.when(w < 8)
        def _():
            base = w * spt
            pltpu.sync_copy(tag_hbm.at[pl.ds(base, LANES)], tg_v)
            tg16 = tg_v[...]
            widx_v[...] = jnp.maximum(tg16, 0)
            pltpu.async_copy(x_hbm.at[widx_v], xg_v, sem).wait()
            pltpu.async_copy(emb_in_hbm.at[pl.ds(base, LANES)], eg_v,
                             sem).wait()

            @pl.loop(0, LANES)
            def _(r):
                m = plsc.load_gather(
                    tg_v, [jnp.full((LANES,), r, jnp.int32)]) >= 0
                for l in range(D // LANES):
                    sl = pl.ds(l * LANES, LANES)
                    xg_v.at[r, sl][...] = jnp.where(
                        m, xg_v.at[r, sl][...], eg_v.at[r, sl][...])

            pltpu.sync_copy(xg_v, emb_ref.at[pl.ds(base, LANES)])

    return k4


def kernel(x, target_id, emb, emb_idx, cached_nodes):
    B, D = x.shape
    num_cache = emb.shape[0]
    num_emb = cached_nodes.shape[0]

    bpt = B // TILES
    nchunk = bpt // 128
    own = ((num_cache + TILES - 1) // TILES + 15) // 16 * 16
    tag_len = own * TILES
    nwords = num_emb // 4
    wpt = ((nwords + TILES - 1) // TILES + 15) // 16 * 16
    nwords_p = wpt * TILES

    cn_u8 = cached_nodes.astype(jnp.uint8)
    cn_words = lax.bitcast_convert_type(cn_u8.reshape(-1, 4), jnp.int32)
    cn_words_p = jnp.pad(cn_words, (0, nwords_p - nwords))

    new_emb0 = _k1_copy(emb)
    out, eidx_all = _make_k2(B, D, bpt, nchunk)(
        target_id, emb_idx, cn_words_p, emb, x)
    tag = _make_kw(B, own, tag_len)(eidx_all)
    ncw = _make_kf(B, nwords_p, wpt)(cn_words_p, target_id, eidx_all)

    emb_ref = jax.new_ref(new_emb0)
    _make_k3(B, D, bpt, nchunk)(eidx_all, tag, x, emb_ref)
    _make_k4(D)(tag, x, emb, emb_ref)
    new_emb = jax.freeze(emb_ref)

    new_cn = lax.bitcast_convert_type(
        ncw[:nwords], jnp.uint8).reshape(num_emb) != 0
    return out, new_emb, new_cn


# speedup vs baseline: 3.6381x; 1.8703x over previous
import jax
import jax.numpy as jnp
from jax import lax
from jax.experimental import pallas as pl
from jax.experimental.pallas import tpu as pltpu
from jax.experimental.pallas import tpu_sc as plsc

NUM_CORES = 2
NUM_SUBCORES = 16
LANES = 16
TILES = NUM_CORES * NUM_SUBCORES

_MESH = dict(core_axis_name="c", subcore_axis_name="s")
_CP = pltpu.CompilerParams(needs_layout_passes=False, use_tc_tiling_on_sc=False)


def _wid():
    return lax.axis_index("s") * NUM_CORES + lax.axis_index("c")


def _iota16():
    return lax.iota(jnp.int32, LANES)


def _make_k2(B, D, bpt, nchunk):
    nb16 = bpt // LANES

    @pl.kernel(
        out_type=(
            jax.ShapeDtypeStruct((B, D), jnp.float32),
            jax.ShapeDtypeStruct((B,), jnp.int32),
        ),
        mesh=plsc.VectorSubcoreMesh(**_MESH),
        scratch_types=[
            pltpu.VMEM((bpt,), jnp.int32),
            pltpu.VMEM((bpt,), jnp.int32),
            pltpu.VMEM((bpt,), jnp.int32),
            pltpu.VMEM((nchunk, 128), jnp.int32),
            pltpu.VMEM((bpt, D), jnp.float32),
            pltpu.VMEM((bpt, D), jnp.float32),
            pltpu.SemaphoreType.DMA,
        ],
        compiler_params=_CP,
    )
    def k2(t_hbm, comb_hbm, x_hbm, emb_ref, out_hbm, eout_hbm,
           t_v, cm_v, ic_v, safe_v, rows_v, x_v, sem):
        w = _wid()
        base = w * bpt
        pltpu.sync_copy(t_hbm.at[pl.ds(base, bpt)], t_v)
        pltpu.async_copy(x_hbm.at[pl.ds(base, bpt)], x_v, sem).wait()

        for c in range(nchunk):
            pltpu.async_copy(comb_hbm.at[t_v.at[pl.ds(c * 128, 128)]],
                             cm_v.at[pl.ds(c * 128, 128)], sem).wait()

        @pl.loop(0, nb16)
        def _(c):
            off = c * LANES
            cm16 = cm_v[pl.ds(off, LANES)]
            e16 = cm16 >> 1
            ic = cm16 & 1
            bglob = base + off + _iota16()
            safe = jnp.where(ic != 0, jnp.maximum(e16, 0), bglob & 0x3FFF)
            cm_v.at[pl.ds(off, LANES)][...] = e16
            ic_v.at[pl.ds(off, LANES)][...] = ic
            ci = c // 8
            cl = (c % 8) * LANES
            safe_v.at[ci, pl.ds(cl, LANES)][...] = safe

        for c in range(nchunk):
            pltpu.async_copy(emb_ref.at[safe_v.at[c]],
                             rows_v.at[pl.ds(c * 128, 128)], sem).wait()

        @pl.loop(0, bpt)
        def _(r):
            m = plsc.load_gather(ic_v, [jnp.full((LANES,), r, jnp.int32)]) != 0
            for l in range(D // LANES):
                sl = pl.ds(l * LANES, LANES)
                rows_v.at[r, sl][...] = jnp.where(m, rows_v.at[r, sl][...],
                                                  x_v.at[r, sl][...])

        pltpu.sync_copy(rows_v, out_hbm.at[pl.ds(base, bpt)])
        pltpu.sync_copy(cm_v, eout_hbm.at[pl.ds(base, bpt)])

    return k2


def _make_k3(B, D, own):
    nchunks = B // LANES
    cap = own + 2 * 128

    @pl.kernel(
        mesh=plsc.VectorSubcoreMesh(**_MESH),
        scratch_types=[
            pltpu.VMEM((B,), jnp.int32),
            pltpu.VMEM((own,), jnp.int32),
            pltpu.VMEM((cap,), jnp.int32),
            pltpu.VMEM((cap,), jnp.int32),
            pltpu.VMEM((128,), jnp.int32),
            pltpu.VMEM((128,), jnp.int32),
            pltpu.VMEM((128, D), jnp.float32),
            pltpu.SemaphoreType.DMA,
        ],
        compiler_params=_CP,
    )
    def k3(eall_hbm, x_hbm, emb_ref,
           e_v, tag_v, ws_v, wb_v, ss_v, sb_v, xr_v, sem):
        w = _wid()
        base = w * own
        pltpu.sync_copy(eall_hbm, e_v)

        @pl.loop(0, own // LANES)
        def _(c):
            tag_v.at[pl.ds(c * LANES, LANES)][...] = jnp.full(
                (LANES,), -1, jnp.int32)

        @pl.loop(0, nchunks)
        def _(c):
            e16 = e_v[pl.ds(c * LANES, LANES)]
            b16 = c * LANES + _iota16()
            mask = (e16 >= base) & (e16 < base + own)
            off = jnp.where(mask, e16 - base, 0)
            plsc.store_scatter(tag_v, [off], b16, mask=mask)

        def cbody(c, cnt):
            t16 = tag_v[pl.ds(c * LANES, LANES)]
            m = t16 >= 0
            s16 = base + c * LANES + _iota16()
            plsc.store_compressed(ws_v.at[pl.ds(cnt, LANES)], s16, mask=m)
            plsc.store_compressed(wb_v.at[pl.ds(cnt, LANES)], t16, mask=m)
            return cnt + jnp.max(plsc.all_reduce_population_count(m))

        cnt = lax.fori_loop(0, own // LANES, cbody, jnp.int32(0))

        @pl.when(cnt > 0)
        def _():
            lasts = plsc.load_gather(
                ws_v, [jnp.full((LANES,), cnt - 1, jnp.int32)])
            lastb = plsc.load_gather(
                wb_v, [jnp.full((LANES,), cnt - 1, jnp.int32)])
            for k in range(8):
                ws_v.at[pl.ds(cnt + k * LANES, LANES)][...] = lasts
                wb_v.at[pl.ds(cnt + k * LANES, LANES)][...] = lastb

            def sbody(g, _):
                @pl.loop(0, 8)
                def _(c):
                    sl = pl.ds(c * LANES, LANES)
                    ss_v.at[sl][...] = ws_v[pl.ds(g * 128 + c * LANES, LANES)]
                    sb_v.at[sl][...] = wb_v[pl.ds(g * 128 + c * LANES, LANES)]
                pltpu.async_copy(x_hbm.at[sb_v], xr_v, sem).wait()
                pltpu.async_copy(xr_v, emb_ref.at[ss_v], sem).wait()
                return 0

            lax.fori_loop(0, (cnt + 127) // 128, sbody, 0)

    return k3


def _make_kf(B, fpt):
    nchunks = B // LANES
    wpt = fpt // 4

    @pl.kernel(
        out_type=jax.ShapeDtypeStruct((wpt * TILES,), jnp.int32),
        mesh=plsc.VectorSubcoreMesh(**_MESH),
        scratch_types=[
            pltpu.VMEM((fpt,), jnp.int32),
            pltpu.VMEM((B,), jnp.int32),
            pltpu.VMEM((B,), jnp.int32),
            pltpu.VMEM((wpt,), jnp.int32),
        ],
        compiler_params=_CP,
    )
    def kf(comb_hbm, t_hbm, eall_hbm, ncw_hbm, c_v, t_v, e_v, w_v):
        w = _wid()
        base = w * fpt
        pltpu.sync_copy(comb_hbm.at[pl.ds(base, fpt)], c_v)
        pltpu.sync_copy(t_hbm, t_v)
        pltpu.sync_copy(eall_hbm, e_v)

        @pl.loop(0, fpt // LANES)
        def _(c):
            sl = pl.ds(c * LANES, LANES)
            c_v.at[sl][...] = c_v.at[sl][...] & 1

        ones = jnp.full((LANES,), 1, jnp.int32)

        @pl.loop(0, nchunks)
        def _(c):
            t16 = t_v[pl.ds(c * LANES, LANES)]
            e16 = e_v[pl.ds(c * LANES, LANES)]
            mask = (e16 != -1) & (t16 >= base) & (t16 < base + fpt)
            off = jnp.where(mask, t16 - base, 0)
            plsc.store_scatter(c_v, [off], ones, mask=mask)

        @pl.loop(0, wpt // LANES)
        def _(j):
            i0 = (j * LANES + _iota16()) * 4
            wd = (plsc.load_gather(c_v, [i0])
                  | (plsc.load_gather(c_v, [i0 + 1]) << 8)
                  | (plsc.load_gather(c_v, [i0 + 2]) << 16)
                  | (plsc.load_gather(c_v, [i0 + 3]) << 24))
            w_v.at[pl.ds(j * LANES, LANES)][...] = wd

        pltpu.sync_copy(w_v, ncw_hbm.at[pl.ds(w * wpt, wpt)])

    return kf


def kernel(x, target_id, emb, emb_idx, cached_nodes):
    B, D = x.shape
    num_cache = emb.shape[0]
    num_emb = cached_nodes.shape[0]

    bpt = B // TILES
    nchunk = bpt // 128
    own = ((num_cache + TILES - 1) // TILES + 15) // 16 * 16
    fpt = ((num_emb + TILES - 1) // TILES + 63) // 64 * 64
    nflags_p = fpt * TILES

    comb = (emb_idx << 1) | cached_nodes.astype(jnp.int32)
    comb_p = jnp.pad(comb, (0, nflags_p - num_emb))

    emb_ref = jax.new_ref(emb)
    out, eidx_all = _make_k2(B, D, bpt, nchunk)(target_id, comb_p, x, emb_ref)
    ncw = _make_kf(B, fpt)(comb_p, target_id, eidx_all)
    _make_k3(B, D, own)(eidx_all, x, emb_ref)
    new_emb = jax.freeze(emb_ref)

    new_cn = lax.bitcast_convert_type(
        ncw[:num_emb // 4], jnp.uint8).reshape(num_emb) != 0
    return out, new_emb, new_cn
